# Initial kernel scaffold; baseline (speedup 1.0000x reference)
import jax
import jax.numpy as jnp
from jax.experimental import pallas as pl


def kernel(x, edge_index, rel_type, norm, W_in_bases, a_in, bias_in, W_h_bases, a_h, bias_h, W_o_bases, a_o, bias_o, gate_W, gate_b):
    raise NotImplementedError("write your pallas kernel here")


# speedup vs baseline: 3.4256x; 3.4256x over previous
import functools

import jax
import jax.numpy as jnp
from jax import lax
from jax.experimental import pallas as pl
from jax.experimental.pallas import tpu as pltpu
from jax.experimental.pallas import tpu_sc as plsc

N = 10000
E = 320000
R = 8
H = 128

NW = 32
CH = 128
CPW = 80
EPAD = NW * CPW * CH
ACC_ROWS = 10240
DUMMY = N

_NBLK = 400
_NGRID = N // _NBLK


def _gidx_body(rel_ref, src_ref, o_ref):
    o_ref[...] = rel_ref[...] * N + src_ref[...]


def _table_in_body(a_ref, bases_ref, o_ref):
    o_ref[...] = jnp.tensordot(a_ref[...], bases_ref[...], axes=[[1], [0]],
                               preferred_element_type=jnp.float32)


def _layer_body(p_ref, bias_ref, a_ref, wb_ref, o_ref):
    h = jnp.maximum(p_ref[0] + p_ref[1] + bias_ref[...], 0.0)
    w = jnp.tensordot(a_ref[...], wb_ref[...], axes=[[1], [0]],
                      preferred_element_type=jnp.float32)
    for r in range(R):
        o_ref[r] = jnp.dot(h, w[r], preferred_element_type=jnp.float32)


def _pool_body(p_ref, bias_ref, gw_ref, gb_ref, o_ref):
    h = p_ref[0] + p_ref[1] + bias_ref[...]
    logits = jnp.sum(h * gw_ref[...], axis=1, keepdims=True) + gb_ref[0, 0]
    m = jnp.max(logits)
    e = jnp.exp(logits - m)
    z = jnp.sum(e)
    o_ref[...] = jnp.sum(e * h, axis=0, keepdims=True) / z


def _tc_gidx(rel2d, src2d):
    return pl.pallas_call(
        _gidx_body,
        out_shape=jax.ShapeDtypeStruct((E // 128, 128), jnp.int32),
    )(rel2d, src2d)


def _tc_table_in(a_in, bases):
    return pl.pallas_call(
        _table_in_body,
        grid=(_NGRID,),
        in_specs=[
            pl.BlockSpec((R, 4), lambda i: (0, 0)),
            pl.BlockSpec((4, _NBLK, H), lambda i: (0, i, 0)),
        ],
        out_specs=pl.BlockSpec((R, _NBLK, H), lambda i: (0, i, 0)),
        out_shape=jax.ShapeDtypeStruct((R, N, H), jnp.float32),
    )(a_in, bases)


def _tc_layer(parts, bias2d, a, wbases):
    return pl.pallas_call(
        _layer_body,
        grid=(_NGRID,),
        in_specs=[
            pl.BlockSpec((2, _NBLK, H), lambda i: (0, i, 0)),
            pl.BlockSpec((1, H), lambda i: (0, 0)),
            pl.BlockSpec((R, 4), lambda i: (0, 0)),
            pl.BlockSpec((4, H, H), lambda i: (0, 0, 0)),
        ],
        out_specs=pl.BlockSpec((R, _NBLK, H), lambda i: (0, i, 0)),
        out_shape=jax.ShapeDtypeStruct((R, N, H), jnp.float32),
    )(parts, bias2d, a, wbases)


def _tc_pool(parts, bias2d, gw2d, gb2d):
    return pl.pallas_call(
        _pool_body,
        grid=(1,),
        in_specs=[
            pl.BlockSpec((2, N, H), lambda i: (0, 0, 0)),
            pl.BlockSpec((1, H), lambda i: (0, 0)),
            pl.BlockSpec((1, H), lambda i: (0, 0)),
            pl.BlockSpec((1, 1), lambda i: (0, 0)),
        ],
        out_specs=pl.BlockSpec((1, H), lambda i: (0, 0)),
        out_shape=jax.ShapeDtypeStruct((1, H), jnp.float32),
    )(parts, bias2d, gw2d, gb2d)


def _sc_scale_rows(rows_ref, norm_ref, c):
    def gbody(g, _):
        nv = norm_ref[c, pl.ds(16 * g, 16)]
        ridx = lax.iota(jnp.int32, 16) + 16 * g
        def cbody(k, _):
            for u in range(8):
                col = k * 8 + u
                cv = jnp.full((16,), col, jnp.int32)
                v = plsc.load_gather(rows_ref, [ridx, cv])
                plsc.store_scatter(rows_ref, [ridx, cv], v * nv)
            return 0
        lax.fori_loop(0, 16, cbody, 0)
        return 0
    lax.fori_loop(0, 8, gbody, 0)


def _sc_edge_body(table, gidxh, dsth, normh, out,
                  gidx_v, dst_v, norm_v, rows0, acc, gsem):
    cid = lax.axis_index("c")
    sid = lax.axis_index("s")
    w = sid * 2 + cid

    pltpu.sync_copy(gidxh.at[pl.ds(w * CPW, CPW)], gidx_v)
    pltpu.sync_copy(dsth.at[pl.ds(w * CPW, CPW)], dst_v)
    pltpu.sync_copy(normh.at[pl.ds(w * CPW, CPW)], norm_v)

    zero = jnp.zeros((16,), jnp.float32)
    def zbody(i, _):
        for j in range(8):
            rows0[i, pl.ds(16 * j, 16)] = zero
        return 0
    lax.fori_loop(0, CH, zbody, 0)
    base = sid * (ACC_ROWS // 16)
    for k in range(5):
        pltpu.sync_copy(rows0, acc.at[pl.ds(base + k * CH, CH)])
    plsc.subcore_barrier()

    def chunk(c, _):
        pltpu.async_copy(table.at[gidx_v.at[c]], rows0, gsem).wait()
        _sc_scale_rows(rows0, norm_v, c)
        pltpu.sync_copy(rows0, acc.at[dst_v.at[c]], add=True)
        return 0
    lax.fori_loop(0, CPW, chunk, 0)
    plsc.subcore_barrier()

    rpt = ACC_ROWS // 16
    pltpu.sync_copy(acc.at[pl.ds(sid * rpt, rpt)],
                    out.at[cid, pl.ds(sid * rpt, rpt)])


def _sc_edge_pass(table2d, gidx2d, dst2d, norm2d):
    mesh = plsc.VectorSubcoreMesh(core_axis_name="c", subcore_axis_name="s",
                                  num_cores=2, num_subcores=16)
    f = functools.partial(
        pl.kernel,
        out_type=jax.ShapeDtypeStruct((2, ACC_ROWS, H), jnp.float32),
        mesh=mesh,
        compiler_params=pltpu.CompilerParams(needs_layout_passes=False),
        scratch_types=[
            pltpu.VMEM((CPW, CH), jnp.int32),
            pltpu.VMEM((CPW, CH), jnp.int32),
            pltpu.VMEM((CPW, CH), jnp.float32),
            pltpu.VMEM((CH, H), jnp.float32),
            pltpu.VMEM_SHARED((ACC_ROWS, H), jnp.float32),
            pltpu.SemaphoreType.DMA,
        ],
    )(_sc_edge_body)
    return f(table2d, gidx2d, dst2d, norm2d)


def kernel(x, edge_index, rel_type, norm,
           W_in_bases, a_in, bias_in,
           W_h_bases, a_h, bias_h,
           W_o_bases, a_o, bias_o,
           gate_W, gate_b):
    src = edge_index[0]
    dst = edge_index[1]

    gidx2d = _tc_gidx(rel_type.reshape(E // 128, 128),
                      src.reshape(E // 128, 128))
    npadr = (EPAD - E) // 128
    gidx_p = jnp.concatenate(
        [gidx2d, jnp.zeros((npadr, 128), jnp.int32)], axis=0)
    dst_p = jnp.concatenate(
        [dst.reshape(E // 128, 128),
         jnp.full((npadr, 128), DUMMY, jnp.int32)], axis=0)
    norm_p = jnp.concatenate(
        [norm.reshape(E // 128, 128),
         jnp.zeros((npadr, 128), jnp.float32)], axis=0)

    t1 = _tc_table_in(a_in, W_in_bases).reshape(R * N, H)
    p1 = _sc_edge_pass(t1, gidx_p, dst_p, norm_p)

    t2 = _tc_layer(p1, bias_in.reshape(1, H), a_h, W_h_bases).reshape(R * N, H)
    p2 = _sc_edge_pass(t2, gidx_p, dst_p, norm_p)

    t3 = _tc_layer(p2, bias_h.reshape(1, H), a_o, W_o_bases).reshape(R * N, H)
    p3 = _sc_edge_pass(t3, gidx_p, dst_p, norm_p)

    return _tc_pool(p3, bias_o.reshape(1, H),
                    gate_W.reshape(1, H), gate_b.reshape(1, 1))


# speedup vs baseline: 3.4673x; 1.0122x over previous
import functools

import jax
import jax.numpy as jnp
from jax import lax
from jax.experimental import pallas as pl
from jax.experimental.pallas import tpu as pltpu
from jax.experimental.pallas import tpu_sc as plsc

N = 10000
E = 320000
R = 8
H = 128

NW = 32
CH = 64
EPAD = 327680
ACC_ROWS = 10240
DUMMY = N

_NBLK = 400
_NGRID = N // _NBLK


def _gidx_body(rel_ref, src_ref, o_ref):
    o_ref[...] = rel_ref[...] * N + src_ref[...]


def _table_in_body(a_ref, bases_ref, o_ref):
    o_ref[...] = jnp.tensordot(a_ref[...], bases_ref[...], axes=[[1], [0]],
                               preferred_element_type=jnp.float32)


def _layer_body(p_ref, bias_ref, a_ref, wb_ref, o_ref):
    h = jnp.maximum(p_ref[0] + p_ref[1] + bias_ref[...], 0.0)
    w = jnp.tensordot(a_ref[...], wb_ref[...], axes=[[1], [0]],
                      preferred_element_type=jnp.float32)
    for r in range(R):
        o_ref[r] = jnp.dot(h, w[r], preferred_element_type=jnp.float32)


def _pool_body(p_ref, bias_ref, gw_ref, gb_ref, o_ref):
    h = p_ref[0] + p_ref[1] + bias_ref[...]
    logits = jnp.sum(h * gw_ref[...], axis=1, keepdims=True) + gb_ref[0, 0]
    m = jnp.max(logits)
    e = jnp.exp(logits - m)
    z = jnp.sum(e)
    o_ref[...] = jnp.sum(e * h, axis=0, keepdims=True) / z


def _tc_gidx(rel2d, src2d):
    return pl.pallas_call(
        _gidx_body,
        out_shape=jax.ShapeDtypeStruct((E // 128, 128), jnp.int32),
    )(rel2d, src2d)


def _tc_table_in(a_in, bases):
    return pl.pallas_call(
        _table_in_body,
        grid=(_NGRID,),
        in_specs=[
            pl.BlockSpec((R, 4), lambda i: (0, 0)),
            pl.BlockSpec((4, _NBLK, H), lambda i: (0, i, 0)),
        ],
        out_specs=pl.BlockSpec((R, _NBLK, H), lambda i: (0, i, 0)),
        out_shape=jax.ShapeDtypeStruct((R, N, H), jnp.float32),
    )(a_in, bases)


def _tc_layer(parts, bias2d, a, wbases):
    return pl.pallas_call(
        _layer_body,
        grid=(_NGRID,),
        in_specs=[
            pl.BlockSpec((2, _NBLK, H), lambda i: (0, i, 0)),
            pl.BlockSpec((1, H), lambda i: (0, 0)),
            pl.BlockSpec((R, 4), lambda i: (0, 0)),
            pl.BlockSpec((4, H, H), lambda i: (0, 0, 0)),
        ],
        out_specs=pl.BlockSpec((R, _NBLK, H), lambda i: (0, i, 0)),
        out_shape=jax.ShapeDtypeStruct((R, N, H), jnp.float32),
    )(parts, bias2d, a, wbases)


def _tc_pool(parts, bias2d, gw2d, gb2d):
    return pl.pallas_call(
        _pool_body,
        grid=(1,),
        in_specs=[
            pl.BlockSpec((2, N, H), lambda i: (0, 0, 0)),
            pl.BlockSpec((1, H), lambda i: (0, 0)),
            pl.BlockSpec((1, H), lambda i: (0, 0)),
            pl.BlockSpec((1, 1), lambda i: (0, 0)),
        ],
        out_specs=pl.BlockSpec((1, H), lambda i: (0, 0)),
        out_shape=jax.ShapeDtypeStruct((1, H), jnp.float32),
    )(parts, bias2d, gw2d, gb2d)


NBUF = 4
NHALF = 4
CPH = EPAD // NW // CH // NHALF


def _sc_scale_rows(rows_ref, norm_ref, c):
    def gbody(g, _):
        nv = norm_ref[c, pl.ds(16 * g, 16)]
        ridx = lax.iota(jnp.int32, 16) + 16 * g
        for col in range(H):
            cv = jnp.full((16,), col, jnp.int32)
            v = plsc.load_gather(rows_ref, [ridx, cv])
            plsc.store_scatter(rows_ref, [ridx, cv], v * nv)
        return 0
    lax.fori_loop(0, CH // 16, gbody, 0)


def _sc_edge_body(table, gidxh, dsth, normh, out,
                  gidx_v, dst_v, norm_v, rows, acc, gsems, ssems):
    cid = lax.axis_index("c")
    sid = lax.axis_index("s")
    w = sid * 2 + cid

    zero = jnp.zeros((16,), jnp.float32)
    def zbody(i, _):
        for j in range(8):
            rows[0][i, pl.ds(16 * j, 16)] = zero
        return 0
    lax.fori_loop(0, CH, zbody, 0)
    base = sid * (ACC_ROWS // 16)
    for k in range(ACC_ROWS // 16 // CH):
        pltpu.sync_copy(rows[0], acc.at[pl.ds(base + k * CH, CH)])
    plsc.subcore_barrier()

    def half(hf, _):
        hb = w * NHALF * CPH + hf * CPH
        pltpu.sync_copy(gidxh.at[pl.ds(hb, CPH)], gidx_v)
        pltpu.sync_copy(dsth.at[pl.ds(hb, CPH)], dst_v)
        pltpu.sync_copy(normh.at[pl.ds(hb, CPH)], norm_v)

        for b in range(NBUF):
            pltpu.async_copy(table.at[gidx_v.at[b]], rows[b], gsems[b])

        def group(gi, _):
            c0 = gi * NBUF
            for b in range(NBUF):
                c = c0 + b
                pltpu.make_async_copy(table.at[gidx_v.at[c]], rows[b],
                                      gsems[b]).wait()
                _sc_scale_rows(rows[b], norm_v, c)
                pltpu.async_copy(rows[b], acc.at[dst_v.at[c]], ssems[b],
                                 add=True)
            @pl.when(gi < CPH // NBUF - 1)
            def _():
                for b in range(NBUF):
                    c = c0 + b
                    pltpu.make_async_copy(rows[b], acc.at[dst_v.at[c]],
                                          ssems[b]).wait()
                    pltpu.async_copy(table.at[gidx_v.at[c + NBUF]], rows[b],
                                     gsems[b])
            return 0
        lax.fori_loop(0, CPH // NBUF, group, 0)
        for b in range(NBUF):
            c = CPH - NBUF + b
            pltpu.make_async_copy(rows[b], acc.at[dst_v.at[c]],
                                  ssems[b]).wait()
        return 0
    lax.fori_loop(0, NHALF, half, 0)
    plsc.subcore_barrier()

    rpt = ACC_ROWS // 16
    pltpu.sync_copy(acc.at[pl.ds(sid * rpt, rpt)],
                    out.at[cid, pl.ds(sid * rpt, rpt)])


def _sc_edge_pass(table2d, gidx2d, dst2d, norm2d):
    mesh = plsc.VectorSubcoreMesh(core_axis_name="c", subcore_axis_name="s",
                                  num_cores=2, num_subcores=16)
    f = functools.partial(
        pl.kernel,
        out_type=jax.ShapeDtypeStruct((2, ACC_ROWS, H), jnp.float32),
        mesh=mesh,
        compiler_params=pltpu.CompilerParams(needs_layout_passes=False),
        scratch_types=[
            pltpu.VMEM((CPH, CH), jnp.int32),
            pltpu.VMEM((CPH, CH), jnp.int32),
            pltpu.VMEM((CPH, CH), jnp.float32),
            tuple(pltpu.VMEM((CH, H), jnp.float32) for _ in range(NBUF)),
            pltpu.VMEM_SHARED((ACC_ROWS, H), jnp.float32),
            tuple(pltpu.SemaphoreType.DMA for _ in range(NBUF)),
            tuple(pltpu.SemaphoreType.DMA for _ in range(NBUF)),
        ],
    )(_sc_edge_body)
    return f(table2d, gidx2d, dst2d, norm2d)


def kernel(x, edge_index, rel_type, norm,
           W_in_bases, a_in, bias_in,
           W_h_bases, a_h, bias_h,
           W_o_bases, a_o, bias_o,
           gate_W, gate_b):
    src = edge_index[0]
    dst = edge_index[1]

    gidx2d = _tc_gidx(rel_type.reshape(E // 128, 128),
                      src.reshape(E // 128, 128))
    npadr = (EPAD - E) // 128
    gidx_p = jnp.concatenate(
        [gidx2d, jnp.zeros((npadr, 128), jnp.int32)], axis=0
    ).reshape(EPAD // CH, CH)
    dst_p = jnp.concatenate(
        [dst.reshape(E // 128, 128),
         jnp.full((npadr, 128), DUMMY, jnp.int32)], axis=0
    ).reshape(EPAD // CH, CH)
    norm_p = jnp.concatenate(
        [norm.reshape(E // 128, 128),
         jnp.zeros((npadr, 128), jnp.float32)], axis=0
    ).reshape(EPAD // CH, CH)

    t1 = _tc_table_in(a_in, W_in_bases).reshape(R * N, H)
    p1 = _sc_edge_pass(t1, gidx_p, dst_p, norm_p)

    t2 = _tc_layer(p1, bias_in.reshape(1, H), a_h, W_h_bases).reshape(R * N, H)
    p2 = _sc_edge_pass(t2, gidx_p, dst_p, norm_p)

    t3 = _tc_layer(p2, bias_h.reshape(1, H), a_o, W_o_bases).reshape(R * N, H)
    p3 = _sc_edge_pass(t3, gidx_p, dst_p, norm_p)

    return _tc_pool(p3, bias_o.reshape(1, H),
                    gate_W.reshape(1, H), gate_b.reshape(1, 1))


# speedup vs baseline: 12.6986x; 3.6624x over previous
import functools

import jax
import jax.numpy as jnp
from jax import lax
from jax.experimental import pallas as pl
from jax.experimental.pallas import tpu as pltpu
from jax.experimental.pallas import tpu_sc as plsc

N = 10000
E = 320000
R = 8
H = 128

NW = 32
CH = 64
EPAD = 327680
ACC_ROWS = 10240
DUMMY = N

_NBLK = 400
_NGRID = N // _NBLK


def _gidx_body(rel_ref, src_ref, o_ref):
    o_ref[...] = rel_ref[...] * N + src_ref[...]


def _table_in_body(a_ref, bases_ref, o_ref):
    o_ref[...] = jnp.tensordot(a_ref[...], bases_ref[...], axes=[[1], [0]],
                               preferred_element_type=jnp.float32)


def _layer_body(p_ref, bias_ref, a_ref, wb_ref, o_ref):
    h = jnp.maximum(p_ref[0] + p_ref[1] + bias_ref[...], 0.0)
    w = jnp.tensordot(a_ref[...], wb_ref[...], axes=[[1], [0]],
                      preferred_element_type=jnp.float32)
    for r in range(R):
        o_ref[r] = jnp.dot(h, w[r], preferred_element_type=jnp.float32)


def _pool_body(p_ref, bias_ref, gw_ref, gb_ref, o_ref):
    h = p_ref[0] + p_ref[1] + bias_ref[...]
    logits = jnp.sum(h * gw_ref[...], axis=1, keepdims=True) + gb_ref[0, 0]
    m = jnp.max(logits)
    e = jnp.exp(logits - m)
    z = jnp.sum(e)
    o_ref[...] = jnp.sum(e * h, axis=0, keepdims=True) / z


def _tc_gidx(rel2d, src2d):
    return pl.pallas_call(
        _gidx_body,
        out_shape=jax.ShapeDtypeStruct((E // 128, 128), jnp.int32),
    )(rel2d, src2d)


def _tc_table_in(a_in, bases):
    return pl.pallas_call(
        _table_in_body,
        grid=(_NGRID,),
        in_specs=[
            pl.BlockSpec((R, 4), lambda i: (0, 0)),
            pl.BlockSpec((4, _NBLK, H), lambda i: (0, i, 0)),
        ],
        out_specs=pl.BlockSpec((R, _NBLK, H), lambda i: (0, i, 0)),
        out_shape=jax.ShapeDtypeStruct((R, N, H), jnp.float32),
    )(a_in, bases)


def _tc_layer(parts, bias2d, a, wbases):
    return pl.pallas_call(
        _layer_body,
        grid=(_NGRID,),
        in_specs=[
            pl.BlockSpec((2, _NBLK, H), lambda i: (0, i, 0)),
            pl.BlockSpec((1, H), lambda i: (0, 0)),
            pl.BlockSpec((R, 4), lambda i: (0, 0)),
            pl.BlockSpec((4, H, H), lambda i: (0, 0, 0)),
        ],
        out_specs=pl.BlockSpec((R, _NBLK, H), lambda i: (0, i, 0)),
        out_shape=jax.ShapeDtypeStruct((R, N, H), jnp.float32),
    )(parts, bias2d, a, wbases)


def _tc_pool(parts, bias2d, gw2d, gb2d):
    return pl.pallas_call(
        _pool_body,
        grid=(1,),
        in_specs=[
            pl.BlockSpec((2, N, H), lambda i: (0, 0, 0)),
            pl.BlockSpec((1, H), lambda i: (0, 0)),
            pl.BlockSpec((1, H), lambda i: (0, 0)),
            pl.BlockSpec((1, 1), lambda i: (0, 0)),
        ],
        out_specs=pl.BlockSpec((1, H), lambda i: (0, 0)),
        out_shape=jax.ShapeDtypeStruct((1, H), jnp.float32),
    )(parts, bias2d, gw2d, gb2d)


NBUF = 4
NHALF = 4
CPH = EPAD // NW // CH // NHALF


def _sc_scale_rows(rows_ref, norm_ref, c):
    def gbody(g, _):
        nv = norm_ref[c, pl.ds(16 * g, 16)]
        for u in range(16):
            e = 16 * g + u
            sp = lax.gather(
                nv, jnp.full((16, 1), u, jnp.int32),
                lax.GatherDimensionNumbers(offset_dims=(),
                                           collapsed_slice_dims=(0,),
                                           start_index_map=(0,)),
                (1,), mode=lax.GatherScatterMode.PROMISE_IN_BOUNDS)
            for j in range(H // 16):
                rows_ref[e, pl.ds(16 * j, 16)] = (
                    rows_ref[e, pl.ds(16 * j, 16)] * sp)
        return 0
    lax.fori_loop(0, CH // 16, gbody, 0)


def _sc_edge_body(table, gidxh, dsth, normh, out,
                  gidx_v, dst_v, norm_v, rows, acc, gsems, ssems):
    cid = lax.axis_index("c")
    sid = lax.axis_index("s")
    w = sid * 2 + cid

    zero = jnp.zeros((16,), jnp.float32)
    def zbody(i, _):
        for j in range(8):
            rows[0][i, pl.ds(16 * j, 16)] = zero
        return 0
    lax.fori_loop(0, CH, zbody, 0)
    base = sid * (ACC_ROWS // 16)
    for k in range(ACC_ROWS // 16 // CH):
        pltpu.sync_copy(rows[0], acc.at[pl.ds(base + k * CH, CH)])
    plsc.subcore_barrier()

    def half(hf, _):
        hb = w * NHALF * CPH + hf * CPH
        pltpu.sync_copy(gidxh.at[pl.ds(hb, CPH)], gidx_v)
        pltpu.sync_copy(dsth.at[pl.ds(hb, CPH)], dst_v)
        pltpu.sync_copy(normh.at[pl.ds(hb, CPH)], norm_v)

        for b in range(NBUF):
            pltpu.async_copy(table.at[gidx_v.at[b]], rows[b], gsems[b])

        def group(gi, _):
            c0 = gi * NBUF
            for b in range(NBUF):
                c = c0 + b
                pltpu.make_async_copy(table.at[gidx_v.at[c]], rows[b],
                                      gsems[b]).wait()
                _sc_scale_rows(rows[b], norm_v, c)
                pltpu.async_copy(rows[b], acc.at[dst_v.at[c]], ssems[b],
                                 add=True)
            @pl.when(gi < CPH // NBUF - 1)
            def _():
                for b in range(NBUF):
                    c = c0 + b
                    pltpu.make_async_copy(rows[b], acc.at[dst_v.at[c]],
                                          ssems[b]).wait()
                    pltpu.async_copy(table.at[gidx_v.at[c + NBUF]], rows[b],
                                     gsems[b])
            return 0
        lax.fori_loop(0, CPH // NBUF, group, 0)
        for b in range(NBUF):
            c = CPH - NBUF + b
            pltpu.make_async_copy(rows[b], acc.at[dst_v.at[c]],
                                  ssems[b]).wait()
        return 0
    lax.fori_loop(0, NHALF, half, 0)
    plsc.subcore_barrier()

    rpt = ACC_ROWS // 16
    pltpu.sync_copy(acc.at[pl.ds(sid * rpt, rpt)],
                    out.at[cid, pl.ds(sid * rpt, rpt)])


def _sc_edge_pass(table2d, gidx2d, dst2d, norm2d):
    mesh = plsc.VectorSubcoreMesh(core_axis_name="c", subcore_axis_name="s",
                                  num_cores=2, num_subcores=16)
    f = functools.partial(
        pl.kernel,
        out_type=jax.ShapeDtypeStruct((2, ACC_ROWS, H), jnp.float32),
        mesh=mesh,
        compiler_params=pltpu.CompilerParams(needs_layout_passes=False),
        scratch_types=[
            pltpu.VMEM((CPH, CH), jnp.int32),
            pltpu.VMEM((CPH, CH), jnp.int32),
            pltpu.VMEM((CPH, CH), jnp.float32),
            tuple(pltpu.VMEM((CH, H), jnp.float32) for _ in range(NBUF)),
            pltpu.VMEM_SHARED((ACC_ROWS, H), jnp.float32),
            tuple(pltpu.SemaphoreType.DMA for _ in range(NBUF)),
            tuple(pltpu.SemaphoreType.DMA for _ in range(NBUF)),
        ],
    )(_sc_edge_body)
    return f(table2d, gidx2d, dst2d, norm2d)


def kernel(x, edge_index, rel_type, norm,
           W_in_bases, a_in, bias_in,
           W_h_bases, a_h, bias_h,
           W_o_bases, a_o, bias_o,
           gate_W, gate_b):
    src = edge_index[0]
    dst = edge_index[1]

    gidx2d = _tc_gidx(rel_type.reshape(E // 128, 128),
                      src.reshape(E // 128, 128))
    npadr = (EPAD - E) // 128
    gidx_p = jnp.concatenate(
        [gidx2d, jnp.zeros((npadr, 128), jnp.int32)], axis=0
    ).reshape(EPAD // CH, CH)
    dst_p = jnp.concatenate(
        [dst.reshape(E // 128, 128),
         jnp.full((npadr, 128), DUMMY, jnp.int32)], axis=0
    ).reshape(EPAD // CH, CH)
    norm_p = jnp.concatenate(
        [norm.reshape(E // 128, 128),
         jnp.zeros((npadr, 128), jnp.float32)], axis=0
    ).reshape(EPAD // CH, CH)

    t1 = _tc_table_in(a_in, W_in_bases).reshape(R * N, H)
    p1 = _sc_edge_pass(t1, gidx_p, dst_p, norm_p)

    t2 = _tc_layer(p1, bias_in.reshape(1, H), a_h, W_h_bases).reshape(R * N, H)
    p2 = _sc_edge_pass(t2, gidx_p, dst_p, norm_p)

    t3 = _tc_layer(p2, bias_h.reshape(1, H), a_o, W_o_bases).reshape(R * N, H)
    p3 = _sc_edge_pass(t3, gidx_p, dst_p, norm_p)

    return _tc_pool(p3, bias_o.reshape(1, H),
                    gate_W.reshape(1, H), gate_b.reshape(1, 1))
